# baseline (speedup 1.0000x reference)
import functools
import jax
import jax.numpy as jnp
from jax import lax
from jax.experimental import pallas as pl
from jax.experimental.pallas import tpu as pltpu
from jax.experimental.pallas import tpu_sc as plsc

F32 = jnp.float32
I32 = jnp.int32

U = 50000
I = 50000
ENT = 100000
REL = 100
DIM = 16
H = 128
NHOP = 2
NMEM = 32
B = 1024
E = 800000

NW = 32
EPT = E // NW
CH = 1000
NCH_E = EPT // CH
UPAD = 50176
ROWS_PER_TILE = UPAD // 16
ZROWS = 392
LAST_ROWS = U - 15 * (UPAD // 16)
RB = 5000
PB = RB // 8
NB = 10
MD = NMEM * DIM


def _leaky(x):
    return jnp.where(x >= 0, x, 0.1 * x)


def _relu(x):
    return jnp.maximum(x, 0.0)


def _sc_mesh():
    return plsc.VectorSubcoreMesh(core_axis_name="c", subcore_axis_name="s")


def _spmm_edges(xu, xi, rows, cols, both):
    _part = jax.ShapeDtypeStruct((2, U, DIM), F32)
    out_types = (_part, _part) if both else _part
    NS = 3

    scratch = [pltpu.VMEM((CH,), I32)] * (2 * NS)
    scratch += [pltpu.VMEM((CH, DIM), F32)] * NS
    scratch += [pltpu.VMEM((ZROWS, DIM), F32)]
    scratch += [pltpu.VMEM_SHARED((UPAD, DIM), F32)]
    scratch += [pltpu.SemaphoreType.DMA] * (2 * NS)

    @functools.partial(pl.kernel, mesh=_sc_mesh(), out_type=out_types,
                       scratch_types=scratch,
                       compiler_params=pltpu.CompilerParams(
                           use_tc_tiling_on_sc=False))
    def spmm_k(xu_hbm, xi_hbm, rows_hbm, cols_hbm, *rest):
        rest = list(rest)
        outs = [rest.pop(0)]
        if both:
            outs.append(rest.pop(0))
        gidx = [rest.pop(0) for _ in range(NS)]
        sidx = [rest.pop(0) for _ in range(NS)]
        gbuf = [rest.pop(0) for _ in range(NS)]
        zbuf = rest.pop(0)
        acc = rest.pop(0)
        isem = [rest.pop(0) for _ in range(NS)]
        gsem = [rest.pop(0) for _ in range(NS)]

        cid = lax.axis_index("c")
        sid = lax.axis_index("s")
        wid = cid * 16 + sid
        ebase = wid * EPT

        zv = jnp.zeros((DIM,), F32)

        def zrow(j, _):
            zbuf[j, :] = zv
            return 0

        def one_pass(tab_hbm, gsrc_hbm, ssrc_hbm, out_hbm):
            lax.fori_loop(0, ZROWS, zrow, 0)
            for j in range(ROWS_PER_TILE // ZROWS):
                off = sid * ROWS_PER_TILE + j * ZROWS
                pltpu.sync_copy(zbuf, acc.at[pl.ds(off, ZROWS)])
            plsc.subcore_barrier()

            def issue_idx(k, s):
                base = ebase + k * CH
                pltpu.async_copy(gsrc_hbm.at[pl.ds(base, CH)], gidx[s],
                                 isem[s])
                pltpu.async_copy(ssrc_hbm.at[pl.ds(base, CH)], sidx[s],
                                 isem[s])

            def wait_idx(s):
                pltpu.make_async_copy(gsrc_hbm.at[pl.ds(0, CH)], gidx[s],
                                      isem[s]).wait()
                pltpu.make_async_copy(ssrc_hbm.at[pl.ds(0, CH)], sidx[s],
                                      isem[s]).wait()

            def issue_gath(s):
                pltpu.async_copy(tab_hbm.at[gidx[s]], gbuf[s], gsem[s])

            def wait_gath(s):
                pltpu.make_async_copy(tab_hbm.at[gidx[s]], gbuf[s],
                                      gsem[s]).wait()

            for j in range(min(NS, NCH_E)):
                issue_idx(j, j)
            for j in range(min(2, NCH_E)):
                wait_idx(j)
                issue_gath(j)

            def step(p, _):
                for b in range(NS):
                    kk = NS * p + b

                    @pl.when(kk < NCH_E)
                    def _():
                        wait_gath(b)
                        pltpu.sync_copy(gbuf[b], acc.at[sidx[b]], add=True)

                        @pl.when(kk + NS < NCH_E)
                        def _():
                            issue_idx(kk + NS, b)

                        @pl.when(kk + 2 < NCH_E)
                        def _():
                            s2 = (b + 2) % NS
                            wait_idx(s2)
                            issue_gath(s2)
                return 0
            lax.fori_loop(0, (NCH_E + NS - 1) // NS, step, 0)

            plsc.subcore_barrier()
            off = sid * ROWS_PER_TILE

            @pl.when(sid < 15)
            def _():
                pltpu.sync_copy(acc.at[pl.ds(off, ROWS_PER_TILE)],
                                out_hbm.at[cid, pl.ds(off, ROWS_PER_TILE)])

            @pl.when(sid == 15)
            def _():
                pltpu.sync_copy(acc.at[pl.ds(15 * ROWS_PER_TILE, LAST_ROWS)],
                                out_hbm.at[cid, pl.ds(15 * ROWS_PER_TILE,
                                                      LAST_ROWS)])

        if both:
            one_pass(xi_hbm, cols_hbm, rows_hbm, outs[0])
        one_pass(xu_hbm, rows_hbm, cols_hbm, outs[-1])

    return spmm_k(xu, xi, rows, cols)


def _sc_gather(table, idx):
    M = idx.shape[0]
    m = M // NW
    ch = min(1024, m)
    nch = m // ch
    NS = 4

    @functools.partial(
        pl.kernel, mesh=_sc_mesh(),
        out_type=jax.ShapeDtypeStruct((M, DIM), F32),
        scratch_types=[pltpu.VMEM((ch,), I32)] * NS
        + [pltpu.VMEM((ch, DIM), F32)] * NS
        + [pltpu.SemaphoreType.DMA] * (2 * NS),
        compiler_params=pltpu.CompilerParams(use_tc_tiling_on_sc=False))
    def gk(tab_hbm, idx_hbm, out_hbm, *rest):
        rest = list(rest)
        iv = [rest.pop(0) for _ in range(NS)]
        gv = [rest.pop(0) for _ in range(NS)]
        isem = [rest.pop(0) for _ in range(NS)]
        gsem = [rest.pop(0) for _ in range(NS)]
        cid = lax.axis_index("c")
        sid = lax.axis_index("s")
        wid = cid * 16 + sid
        tbase = wid * m

        def issue_idx(k, s):
            pltpu.async_copy(idx_hbm.at[pl.ds(tbase + k * ch, ch)],
                             iv[s], isem[s])

        def wait_idx(s):
            pltpu.make_async_copy(idx_hbm.at[pl.ds(0, ch)],
                                  iv[s], isem[s]).wait()

        def issue_gath(s):
            pltpu.async_copy(tab_hbm.at[iv[s]], gv[s], gsem[s])

        def wait_gath(s):
            pltpu.make_async_copy(tab_hbm.at[iv[s]], gv[s], gsem[s]).wait()

        for j in range(min(NS, nch)):
            issue_idx(j, j)
        for j in range(min(2, nch)):
            wait_idx(j)
            issue_gath(j)

        def step(p, _):
            for b in range(NS):
                kk = NS * p + b

                @pl.when(kk < nch)
                def _():
                    wait_gath(b)
                    pltpu.sync_copy(gv[b],
                                    out_hbm.at[pl.ds(tbase + kk * ch, ch)])

                    @pl.when(kk + NS < nch)
                    def _():
                        issue_idx(kk + NS, b)

                    @pl.when(kk + 2 < nch)
                    def _():
                        s2 = (b + 2) % NS
                        wait_idx(s2)
                        issue_gath(s2)
            return 0
        lax.fori_loop(0, (nch + NS - 1) // NS, step, 0)

    return gk(table, idx)


def _dotT(a, b):
    return lax.dot_general(a, b, (((1,), (1,)), ((), ())),
                           preferred_element_type=F32)


def _dot0(a, b):
    return lax.dot_general(a, b, (((0,), (0,)), ((), ())),
                           preferred_element_type=F32)


def _chain(m1, wc):
    m2 = _relu(jnp.dot(m1, wc[0], preferred_element_type=F32)) + m1
    m3 = _relu(jnp.dot(m2, wc[1], preferred_element_type=F32)) + m2
    m4 = _relu(jnp.dot(m3, wc[2], preferred_element_type=F32)) + m3
    return m4


_SPEC_PK = pl.BlockSpec((1, PB, 128), lambda i: (i, 0, 0))
_SPEC_HYPER = pl.BlockSpec((DIM, H), lambda i: (0, 0))
_SPEC_M4 = pl.BlockSpec((DIM, H), lambda i: (0, 0))
_SPEC_CHAIN = pl.BlockSpec((3, H, H), lambda i: (0, 0, 0))
_SPEC_W16 = pl.BlockSpec((DIM, DIM), lambda i: (0, 0))
_SPEC_PART = pl.BlockSpec((2, 1, PB, 128),
                          lambda i: (0, i, 0, 0))
_ARB = pltpu.CompilerParams(dimension_semantics=("arbitrary",))


def _k2_l1(u0p, i0p, uhyper, ihyper, wcu, wci):
    def body(u0r, i0r, wuh, wih, wcu_r, wci_r, m4u_o, m4i_o, accu, acci):
        i = pl.program_id(0)

        @pl.when(i == 0)
        def _():
            accu[...] = jnp.zeros((DIM, H), F32)
            acci[...] = jnp.zeros((DIM, H), F32)

        du = jnp.zeros((DIM, H), F32)
        di = jnp.zeros((DIM, H), F32)
        for g in range(8):
            u0g = u0r[0, :, g * DIM:(g + 1) * DIM]
            i0g = i0r[0, :, g * DIM:(g + 1) * DIM]
            aug = jnp.dot(u0g, wuh[...], preferred_element_type=F32)
            aig = jnp.dot(i0g, wih[...], preferred_element_type=F32)
            du = du + _dot0(u0g, aug)
            di = di + _dot0(i0g, aig)
        accu[...] += du
        acci[...] += di

        @pl.when(i == NB - 1)
        def _():
            m4u_o[...] = _chain(_leaky(accu[...]), wcu_r)
            m4i_o[...] = _chain(_leaky(acci[...]), wci_r)

    return pl.pallas_call(
        body,
        grid=(NB,),
        in_specs=[_SPEC_PK, _SPEC_PK, _SPEC_HYPER, _SPEC_HYPER,
                  _SPEC_CHAIN, _SPEC_CHAIN],
        out_specs=[_SPEC_M4, _SPEC_M4],
        out_shape=[jax.ShapeDtypeStruct((DIM, H), F32)] * 2,
        scratch_shapes=[pltpu.VMEM((DIM, H), F32)] * 2,
        compiler_params=_ARB,
    )(u0p, i0p, uhyper, ihyper, wcu, wci)


def _k2_l2(i0p, z1p, ihyper, wci):
    def body(i0r, z1r, wih, wci_r, m4_o, acc):
        i = pl.program_id(0)

        @pl.when(i == 0)
        def _():
            acc[...] = jnp.zeros((DIM, H), F32)

        d = jnp.zeros((DIM, H), F32)
        for g in range(8):
            i0g = i0r[0, :, g * DIM:(g + 1) * DIM]
            z1g = z1r[0, :, g * DIM:(g + 1) * DIM]
            aig = jnp.dot(i0g, wih[...], preferred_element_type=F32)
            d = d + _dot0(z1g, aig)
        acc[...] += d

        @pl.when(i == NB - 1)
        def _():
            m4_o[...] = _chain(_leaky(acc[...]), wci_r)

    return pl.pallas_call(
        body,
        grid=(NB,),
        in_specs=[_SPEC_PK, _SPEC_PK, _SPEC_HYPER, _SPEC_CHAIN],
        out_specs=_SPEC_M4,
        out_shape=jax.ShapeDtypeStruct((DIM, H), F32),
        scratch_shapes=[pltpu.VMEM((DIM, H), F32)],
        compiler_params=_ARB,
    )(i0p, z1p, ihyper, wci)


def _k4_l1(u0p, i0p, uhyper, ihyper, m4u, m4i, SUp, SIp, wgu, wgi):
    def body(u0r, i0r, wuh, wih, m4u_r, m4i_r, su_r, si_r, wgu_r, wgi_r,
             u1_o, z1_o):
        ug = []
        zg = []
        for g in range(8):
            sl = slice(g * DIM, (g + 1) * DIM)
            u0g = u0r[0, :, sl]
            i0g = i0r[0, :, sl]
            aug = jnp.dot(u0g, wuh[...], preferred_element_type=F32)
            aig = jnp.dot(i0g, wih[...], preferred_element_type=F32)
            hUg = _leaky(_dotT(aug, m4u_r[...]))
            hIg = _leaky(_dotT(aig, m4i_r[...]))
            sUg = su_r[0, 0, :, sl] + su_r[1, 0, :, sl]
            sIg = si_r[0, 0, :, sl] + si_r[1, 0, :, sl]
            gUg = _leaky(_relu(jnp.dot(sUg, wgu_r[...],
                                       preferred_element_type=F32)))
            gIg = _leaky(_relu(jnp.dot(sIg, wgi_r[...],
                                       preferred_element_type=F32)))
            ug.append(hUg + gUg + u0g)
            zg.append(hIg + gIg + i0g)
        u1_o[...] = jnp.concatenate(ug, axis=1)[None]
        z1_o[...] = jnp.concatenate(zg, axis=1)[None]

    return pl.pallas_call(
        body,
        grid=(NB,),
        in_specs=[_SPEC_PK, _SPEC_PK, _SPEC_HYPER, _SPEC_HYPER,
                  _SPEC_M4, _SPEC_M4, _SPEC_PART, _SPEC_PART,
                  _SPEC_W16, _SPEC_W16],
        out_specs=[_SPEC_PK, _SPEC_PK],
        out_shape=[jax.ShapeDtypeStruct((NB, PB, 128), F32)] * 2,
        compiler_params=_ARB,
    )(u0p, i0p, uhyper, ihyper, m4u, m4i, SUp, SIp, wgu, wgi)


def _k_item(i0_pos, z1_pos, s2g, ihyper, m4i2, wgi1, rview):
    def body(i0r, z1r, s2r, wih, m4_r, wg_r, rv_r, it_o, v0_o):
        s2 = s2r[0:B, :] + s2r[B:2 * B, :]
        ai = jnp.dot(i0r[...], wih[...], preferred_element_type=F32)
        hI = _leaky(_dotT(ai, m4_r[...]))
        gI = _leaky(_relu(jnp.dot(s2, wg_r[...],
                                  preferred_element_type=F32)))
        item = i0r[...] + 2.0 * z1r[...] + hI + gI
        it_o[...] = item
        v0_o[...] = _dotT(item, rv_r[...])

    return pl.pallas_call(
        body,
        grid=(1,),
        in_specs=[pl.BlockSpec((B, DIM), lambda i: (0, 0)),
                  pl.BlockSpec((B, DIM), lambda i: (0, 0)),
                  pl.BlockSpec((2 * B, DIM), lambda i: (0, 0)),
                  pl.BlockSpec((DIM, H), lambda i: (0, 0)),
                  pl.BlockSpec((DIM, H), lambda i: (0, 0)),
                  pl.BlockSpec((DIM, DIM), lambda i: (0, 0)),
                  pl.BlockSpec((REL * DIM, DIM), lambda i: (0, 0))],
        out_specs=[pl.BlockSpec((B, DIM), lambda i: (0, 0)),
                   pl.BlockSpec((B, REL * DIM), lambda i: (0, 0))],
        out_shape=[jax.ShapeDtypeStruct((B, DIM), F32),
                   jax.ShapeDtypeStruct((B, REL * DIM), F32)],
    )(i0_pos, z1_pos, s2g, ihyper, m4i2, wgi1, rview)


def _hop_core(vh, h, t0, t1, item):
    jg = lax.broadcasted_iota(jnp.int32, (MD, NMEM), 0) // DIM
    mg = lax.broadcasted_iota(jnp.int32, (MD, NMEM), 1)
    G = (jg == mg).astype(F32)
    ji = lax.broadcasted_iota(jnp.int32, (MD, DIM), 0) % DIM
    ii = lax.broadcasted_iota(jnp.int32, (MD, DIM), 1)
    S = (ji == ii).astype(F32)
    hv = h * vh
    logits = jnp.dot(hv, G, preferred_element_type=F32)
    mx = jnp.max(logits, axis=1, keepdims=True)
    ex = jnp.exp(logits - mx)
    p = ex / jnp.sum(ex, axis=1, keepdims=True)
    pe = _dotT(p, G)
    w = (t0 + t1) * pe
    o = jnp.dot(w, S, preferred_element_type=F32)
    return o, item + o


_SPEC_H = pl.BlockSpec((1, B, MD), lambda i: (0, 0, 0))
_SPEC_H1 = pl.BlockSpec((1, B, MD), lambda i: (1, 0, 0))


def _t_spec(hop, ar):
    return pl.BlockSpec((1, 1, B, MD), lambda i: (hop, ar, 0, 0))


def _k6_hop0(vh, h_arr, t_arr, item, rview):
    def body(vh_r, h_r, t0_r, t1_r, it_r, rv_r, it1_o, o_o, v1_o):
        o, it1 = _hop_core(vh_r[...], h_r[0], t0_r[0, 0], t1_r[0, 0],
                           it_r[...])
        it1_o[...] = it1
        o_o[...] = o
        v1_o[...] = _dotT(it1, rv_r[...])

    return pl.pallas_call(
        body,
        grid=(1,),
        in_specs=[pl.BlockSpec((B, MD), lambda i: (0, 0)), _SPEC_H,
                  _t_spec(0, 0), _t_spec(0, 1),
                  pl.BlockSpec((B, DIM), lambda i: (0, 0)),
                  pl.BlockSpec((REL * DIM, DIM), lambda i: (0, 0))],
        out_specs=[pl.BlockSpec((B, DIM), lambda i: (0, 0)),
                   pl.BlockSpec((B, DIM), lambda i: (0, 0)),
                   pl.BlockSpec((B, REL * DIM), lambda i: (0, 0))],
        out_shape=[jax.ShapeDtypeStruct((B, DIM), F32),
                   jax.ShapeDtypeStruct((B, DIM), F32),
                   jax.ShapeDtypeStruct((B, REL * DIM), F32)],
    )(vh, h_arr, t_arr, t_arr, item, rview)


def _k6_hop1(vh, h_arr, t_arr, item1, o0):
    def body(vh_r, h_r, t0_r, t1_r, it_r, o0_r, sc_o):
        o1, it2 = _hop_core(vh_r[...], h_r[0], t0_r[0, 0], t1_r[0, 0],
                            it_r[...])
        y = o0_r[...] + o1
        s = jax.nn.sigmoid(jnp.sum(it2 * y, axis=1))
        sc_o[...] = s.reshape(8, 128)

    return pl.pallas_call(
        body,
        grid=(1,),
        in_specs=[pl.BlockSpec((B, MD), lambda i: (0, 0)), _SPEC_H1,
                  _t_spec(1, 0), _t_spec(1, 1),
                  pl.BlockSpec((B, DIM), lambda i: (0, 0)),
                  pl.BlockSpec((B, DIM), lambda i: (0, 0))],
        out_specs=pl.BlockSpec((8, 128), lambda i: (0, 0)),
        out_shape=jax.ShapeDtypeStruct((8, 128), F32),
    )(vh, h_arr, t_arr, t_arr, item1, o0)


def kernel(uEmbed0, iEmbed0, uhyper, ihyper, entity_emb, relation_emb,
           W_hgnn_u, W_hgnn_i, W_gcn_u, W_gcn_i, edge_vals,
           edge_index, pos_items, memories_h, memories_r, memories_t):
    rows = edge_index[0]
    cols = edge_index[1]
    u0p = uEmbed0.reshape(NB, PB, 128)
    i0p = iEmbed0.reshape(NB, PB, 128)
    u0v = u0p.reshape(U, DIM)
    i0v = i0p.reshape(I, DIM)

    h_flat = _sc_gather(entity_emb, memories_h.reshape(-1))
    t_flat = _sc_gather(entity_emb, memories_t.reshape(-1))
    i0_pos = _sc_gather(i0v, pos_items)
    rows, cols, h_flat, t_flat, i0_pos = lax.optimization_barrier(
        (rows, cols, h_flat, t_flat, i0_pos))
    h_arr = h_flat.reshape(NHOP, B, MD)
    t_arr = t_flat.reshape(NHOP, 2, B, MD)

    SU1, SI1 = _spmm_edges(u0v, i0v, rows, cols, both=True)
    SU1p = SU1.reshape(2, NB, PB, 128)
    SI1p = SI1.reshape(2, NB, PB, 128)
    m4u, m4i = _k2_l1(u0p, i0p, uhyper, ihyper, W_hgnn_u[0], W_hgnn_i[0])
    u1p, z1p = _k4_l1(u0p, i0p, uhyper, ihyper, m4u, m4i,
                      SU1p, SI1p, W_gcn_u[0], W_gcn_i[0])

    u1 = u1p.reshape(U, DIM)
    z1 = z1p.reshape(I, DIM)
    SI2 = _spmm_edges(u1, u1, rows, cols, both=False)
    m4i2 = _k2_l2(i0p, z1p, ihyper, W_hgnn_i[1])
    z1_pos = _sc_gather(z1, pos_items)
    s2_idx = jnp.concatenate([pos_items, pos_items + U])
    s2g = _sc_gather(SI2.reshape(2 * U, DIM), s2_idx)

    rview = relation_emb.reshape(REL * DIM, DIM)
    item_emb, v0 = _k_item(i0_pos, z1_pos, s2g, ihyper, m4i2,
                           W_gcn_i[1], rview)

    ridx = (memories_r.astype(I32)
            + jnp.arange(B, dtype=I32)[None, :, None] * REL)
    vh0 = _sc_gather(v0.reshape(B * REL, DIM), ridx[0].reshape(-1))
    item1, o0, v1 = _k6_hop0(vh0.reshape(B, MD), h_arr, t_arr,
                             item_emb, rview)
    vh1 = _sc_gather(v1.reshape(B * REL, DIM), ridx[1].reshape(-1))
    scores = _k6_hop1(vh1.reshape(B, MD), h_arr, t_arr, item1, o0)
    return scores.reshape(B)


# speedup vs baseline: 1.2883x; 1.2883x over previous
import functools
import jax
import jax.numpy as jnp
from jax import lax
from jax.experimental import pallas as pl
from jax.experimental.pallas import tpu as pltpu
from jax.experimental.pallas import tpu_sc as plsc

F32 = jnp.float32
I32 = jnp.int32

U = 50000
I = 50000
ENT = 100000
REL = 100
DIM = 16
H = 128
NHOP = 2
NMEM = 32
B = 1024
E = 800000

NW = 32
EPT = E // NW
CH = 1000
NCH_E = EPT // CH
UPAD = 50176
ROWS_PER_TILE = UPAD // 16
ZROWS = 392
LAST_ROWS = U - 15 * (UPAD // 16)
RB = 5000
PB = RB // 8
NB = 10
MD = NMEM * DIM


def _leaky(x):
    return jnp.where(x >= 0, x, 0.1 * x)


def _relu(x):
    return jnp.maximum(x, 0.0)


def _sc_mesh():
    return plsc.VectorSubcoreMesh(core_axis_name="c", subcore_axis_name="s")


def _spmm_edges(xu, xi, rows, cols, both):
    _part = jax.ShapeDtypeStruct((2, U, DIM), F32)
    out_types = (_part, _part) if both else _part
    NS = 3

    scratch = [pltpu.VMEM((CH,), I32)] * (2 * NS)
    scratch += [pltpu.VMEM((CH, DIM), F32)] * NS
    scratch += [pltpu.VMEM((ZROWS, DIM), F32)]
    scratch += [pltpu.VMEM_SHARED((UPAD, DIM), F32)]
    scratch += [pltpu.SemaphoreType.DMA] * (2 * NS)

    @functools.partial(pl.kernel, mesh=_sc_mesh(), out_type=out_types,
                       scratch_types=scratch,
                       compiler_params=pltpu.CompilerParams(
                           use_tc_tiling_on_sc=False))
    def spmm_k(xu_hbm, xi_hbm, rows_hbm, cols_hbm, *rest):
        rest = list(rest)
        outs = [rest.pop(0)]
        if both:
            outs.append(rest.pop(0))
        gidx = [rest.pop(0) for _ in range(NS)]
        sidx = [rest.pop(0) for _ in range(NS)]
        gbuf = [rest.pop(0) for _ in range(NS)]
        zbuf = rest.pop(0)
        acc = rest.pop(0)
        isem = [rest.pop(0) for _ in range(NS)]
        gsem = [rest.pop(0) for _ in range(NS)]

        cid = lax.axis_index("c")
        sid = lax.axis_index("s")
        wid = cid * 16 + sid
        ebase = wid * EPT

        zv = jnp.zeros((DIM,), F32)

        def zrow(j, _):
            zbuf[j, :] = zv
            return 0

        def one_pass(tab_hbm, gsrc_hbm, ssrc_hbm, out_hbm):
            lax.fori_loop(0, ZROWS, zrow, 0)
            for j in range(ROWS_PER_TILE // ZROWS):
                off = sid * ROWS_PER_TILE + j * ZROWS
                pltpu.sync_copy(zbuf, acc.at[pl.ds(off, ZROWS)])
            plsc.subcore_barrier()

            def issue_idx(k, s):
                base = ebase + k * CH
                pltpu.async_copy(gsrc_hbm.at[pl.ds(base, CH)], gidx[s],
                                 isem[s])
                pltpu.async_copy(ssrc_hbm.at[pl.ds(base, CH)], sidx[s],
                                 isem[s])

            def wait_idx(s):
                pltpu.make_async_copy(gsrc_hbm.at[pl.ds(0, CH)], gidx[s],
                                      isem[s]).wait()
                pltpu.make_async_copy(ssrc_hbm.at[pl.ds(0, CH)], sidx[s],
                                      isem[s]).wait()

            def issue_gath(s):
                pltpu.async_copy(tab_hbm.at[gidx[s]], gbuf[s], gsem[s])

            def wait_gath(s):
                pltpu.make_async_copy(tab_hbm.at[gidx[s]], gbuf[s],
                                      gsem[s]).wait()

            for j in range(min(NS, NCH_E)):
                issue_idx(j, j)
            for j in range(min(2, NCH_E)):
                wait_idx(j)
                issue_gath(j)

            def step(p, _):
                for b in range(NS):
                    kk = NS * p + b

                    @pl.when(kk < NCH_E)
                    def _():
                        wait_gath(b)
                        pltpu.sync_copy(gbuf[b], acc.at[sidx[b]], add=True)

                        @pl.when(kk + NS < NCH_E)
                        def _():
                            issue_idx(kk + NS, b)

                        @pl.when(kk + 2 < NCH_E)
                        def _():
                            s2 = (b + 2) % NS
                            wait_idx(s2)
                            issue_gath(s2)
                return 0
            lax.fori_loop(0, (NCH_E + NS - 1) // NS, step, 0)

            plsc.subcore_barrier()
            off = sid * ROWS_PER_TILE

            @pl.when(sid < 15)
            def _():
                pltpu.sync_copy(acc.at[pl.ds(off, ROWS_PER_TILE)],
                                out_hbm.at[cid, pl.ds(off, ROWS_PER_TILE)])

            @pl.when(sid == 15)
            def _():
                pltpu.sync_copy(acc.at[pl.ds(15 * ROWS_PER_TILE, LAST_ROWS)],
                                out_hbm.at[cid, pl.ds(15 * ROWS_PER_TILE,
                                                      LAST_ROWS)])

        if both:
            one_pass(xi_hbm, cols_hbm, rows_hbm, outs[0])
        one_pass(xu_hbm, rows_hbm, cols_hbm, outs[-1])

    return spmm_k(xu, xi, rows, cols)


def _sc_gather(table, idx):
    M = idx.shape[0]
    m = M // NW
    ch = min(1024, m)
    nch = m // ch
    NS = 4

    @functools.partial(
        pl.kernel, mesh=_sc_mesh(),
        out_type=jax.ShapeDtypeStruct((M, DIM), F32),
        scratch_types=[pltpu.VMEM((ch,), I32)] * NS
        + [pltpu.VMEM((ch, DIM), F32)] * NS
        + [pltpu.SemaphoreType.DMA] * (2 * NS),
        compiler_params=pltpu.CompilerParams(use_tc_tiling_on_sc=False))
    def gk(tab_hbm, idx_hbm, out_hbm, *rest):
        rest = list(rest)
        iv = [rest.pop(0) for _ in range(NS)]
        gv = [rest.pop(0) for _ in range(NS)]
        isem = [rest.pop(0) for _ in range(NS)]
        gsem = [rest.pop(0) for _ in range(NS)]
        cid = lax.axis_index("c")
        sid = lax.axis_index("s")
        wid = cid * 16 + sid
        tbase = wid * m

        def issue_idx(k, s):
            pltpu.async_copy(idx_hbm.at[pl.ds(tbase + k * ch, ch)],
                             iv[s], isem[s])

        def wait_idx(s):
            pltpu.make_async_copy(idx_hbm.at[pl.ds(0, ch)],
                                  iv[s], isem[s]).wait()

        def issue_gath(s):
            pltpu.async_copy(tab_hbm.at[iv[s]], gv[s], gsem[s])

        def wait_gath(s):
            pltpu.make_async_copy(tab_hbm.at[iv[s]], gv[s], gsem[s]).wait()

        for j in range(min(NS, nch)):
            issue_idx(j, j)
        for j in range(min(2, nch)):
            wait_idx(j)
            issue_gath(j)

        def step(p, _):
            for b in range(NS):
                kk = NS * p + b

                @pl.when(kk < nch)
                def _():
                    wait_gath(b)
                    pltpu.sync_copy(gv[b],
                                    out_hbm.at[pl.ds(tbase + kk * ch, ch)])

                    @pl.when(kk + NS < nch)
                    def _():
                        issue_idx(kk + NS, b)

                    @pl.when(kk + 2 < nch)
                    def _():
                        s2 = (b + 2) % NS
                        wait_idx(s2)
                        issue_gath(s2)
            return 0
        lax.fori_loop(0, (nch + NS - 1) // NS, step, 0)

    return gk(table, idx)


def _dotT(a, b):
    return lax.dot_general(a, b, (((1,), (1,)), ((), ())),
                           preferred_element_type=F32)


def _dot0(a, b):
    return lax.dot_general(a, b, (((0,), (0,)), ((), ())),
                           preferred_element_type=F32)


def _chain(m1, wc):
    m2 = _relu(jnp.dot(m1, wc[0], preferred_element_type=F32)) + m1
    m3 = _relu(jnp.dot(m2, wc[1], preferred_element_type=F32)) + m2
    m4 = _relu(jnp.dot(m3, wc[2], preferred_element_type=F32)) + m3
    return m4


_SPEC_PK = pl.BlockSpec((1, PB, 128), lambda i: (i, 0, 0))
_SPEC_HYPER = pl.BlockSpec((DIM, H), lambda i: (0, 0))
_SPEC_M4 = pl.BlockSpec((DIM, H), lambda i: (0, 0))
_SPEC_CHAIN = pl.BlockSpec((3, H, H), lambda i: (0, 0, 0))
_SPEC_W16 = pl.BlockSpec((DIM, DIM), lambda i: (0, 0))
_SPEC_PART = pl.BlockSpec((2, 1, PB, 128),
                          lambda i: (0, i, 0, 0))
_ARB = pltpu.CompilerParams(dimension_semantics=("arbitrary",))


def _k2_l1(u0p, i0p, uhyper, ihyper, wcu, wci):
    def body(u0r, i0r, wuh, wih, wcu_r, wci_r, m4u_o, m4i_o, accu, acci):
        i = pl.program_id(0)

        @pl.when(i == 0)
        def _():
            accu[...] = jnp.zeros((DIM, H), F32)
            acci[...] = jnp.zeros((DIM, H), F32)

        du = jnp.zeros((DIM, H), F32)
        di = jnp.zeros((DIM, H), F32)
        for g in range(8):
            u0g = u0r[0, :, g * DIM:(g + 1) * DIM]
            i0g = i0r[0, :, g * DIM:(g + 1) * DIM]
            aug = jnp.dot(u0g, wuh[...], preferred_element_type=F32)
            aig = jnp.dot(i0g, wih[...], preferred_element_type=F32)
            du = du + _dot0(u0g, aug)
            di = di + _dot0(i0g, aig)
        accu[...] += du
        acci[...] += di

        @pl.when(i == NB - 1)
        def _():
            m4u_o[...] = _chain(_leaky(accu[...]), wcu_r)
            m4i_o[...] = _chain(_leaky(acci[...]), wci_r)

    return pl.pallas_call(
        body,
        grid=(NB,),
        in_specs=[_SPEC_PK, _SPEC_PK, _SPEC_HYPER, _SPEC_HYPER,
                  _SPEC_CHAIN, _SPEC_CHAIN],
        out_specs=[_SPEC_M4, _SPEC_M4],
        out_shape=[jax.ShapeDtypeStruct((DIM, H), F32)] * 2,
        scratch_shapes=[pltpu.VMEM((DIM, H), F32)] * 2,
        compiler_params=_ARB,
    )(u0p, i0p, uhyper, ihyper, wcu, wci)


def _k2_l2(i0p, z1p, ihyper, wci):
    def body(i0r, z1r, wih, wci_r, m4_o, acc):
        i = pl.program_id(0)

        @pl.when(i == 0)
        def _():
            acc[...] = jnp.zeros((DIM, H), F32)

        d = jnp.zeros((DIM, H), F32)
        for g in range(8):
            i0g = i0r[0, :, g * DIM:(g + 1) * DIM]
            z1g = z1r[0, :, g * DIM:(g + 1) * DIM]
            aig = jnp.dot(i0g, wih[...], preferred_element_type=F32)
            d = d + _dot0(z1g, aig)
        acc[...] += d

        @pl.when(i == NB - 1)
        def _():
            m4_o[...] = _chain(_leaky(acc[...]), wci_r)

    return pl.pallas_call(
        body,
        grid=(NB,),
        in_specs=[_SPEC_PK, _SPEC_PK, _SPEC_HYPER, _SPEC_CHAIN],
        out_specs=_SPEC_M4,
        out_shape=jax.ShapeDtypeStruct((DIM, H), F32),
        scratch_shapes=[pltpu.VMEM((DIM, H), F32)],
        compiler_params=_ARB,
    )(i0p, z1p, ihyper, wci)


def _k4_l1(u0p, i0p, uhyper, ihyper, m4u, m4i, SUp, SIp, wgu, wgi):
    def body(u0r, i0r, wuh, wih, m4u_r, m4i_r, su_r, si_r, wgu_r, wgi_r,
             u1_o, z1_o):
        ug = []
        zg = []
        for g in range(8):
            sl = slice(g * DIM, (g + 1) * DIM)
            u0g = u0r[0, :, sl]
            i0g = i0r[0, :, sl]
            aug = jnp.dot(u0g, wuh[...], preferred_element_type=F32)
            aig = jnp.dot(i0g, wih[...], preferred_element_type=F32)
            hUg = _leaky(_dotT(aug, m4u_r[...]))
            hIg = _leaky(_dotT(aig, m4i_r[...]))
            sUg = su_r[0, 0, :, sl] + su_r[1, 0, :, sl]
            sIg = si_r[0, 0, :, sl] + si_r[1, 0, :, sl]
            gUg = _leaky(_relu(jnp.dot(sUg, wgu_r[...],
                                       preferred_element_type=F32)))
            gIg = _leaky(_relu(jnp.dot(sIg, wgi_r[...],
                                       preferred_element_type=F32)))
            ug.append(hUg + gUg + u0g)
            zg.append(hIg + gIg + i0g)
        u1_o[...] = jnp.concatenate(ug, axis=1)[None]
        z1_o[...] = jnp.concatenate(zg, axis=1)[None]

    return pl.pallas_call(
        body,
        grid=(NB,),
        in_specs=[_SPEC_PK, _SPEC_PK, _SPEC_HYPER, _SPEC_HYPER,
                  _SPEC_M4, _SPEC_M4, _SPEC_PART, _SPEC_PART,
                  _SPEC_W16, _SPEC_W16],
        out_specs=[_SPEC_PK, _SPEC_PK],
        out_shape=[jax.ShapeDtypeStruct((NB, PB, 128), F32)] * 2,
        compiler_params=_ARB,
    )(u0p, i0p, uhyper, ihyper, m4u, m4i, SUp, SIp, wgu, wgi)


def _k_item(i0_pos, z1_pos, s2g, ihyper, m4i2, wgi1, rview):
    def body(i0r, z1r, s2r, wih, m4_r, wg_r, rv_r, it_o, v0_o):
        s2 = s2r[0:B, :] + s2r[B:2 * B, :]
        ai = jnp.dot(i0r[...], wih[...], preferred_element_type=F32)
        hI = _leaky(_dotT(ai, m4_r[...]))
        gI = _leaky(_relu(jnp.dot(s2, wg_r[...],
                                  preferred_element_type=F32)))
        item = i0r[...] + 2.0 * z1r[...] + hI + gI
        it_o[...] = item
        v0_o[...] = _dotT(item, rv_r[...])

    return pl.pallas_call(
        body,
        grid=(1,),
        in_specs=[pl.BlockSpec((B, DIM), lambda i: (0, 0)),
                  pl.BlockSpec((B, DIM), lambda i: (0, 0)),
                  pl.BlockSpec((2 * B, DIM), lambda i: (0, 0)),
                  pl.BlockSpec((DIM, H), lambda i: (0, 0)),
                  pl.BlockSpec((DIM, H), lambda i: (0, 0)),
                  pl.BlockSpec((DIM, DIM), lambda i: (0, 0)),
                  pl.BlockSpec((REL * DIM, DIM), lambda i: (0, 0))],
        out_specs=[pl.BlockSpec((B, DIM), lambda i: (0, 0)),
                   pl.BlockSpec((B, REL * DIM), lambda i: (0, 0))],
        out_shape=[jax.ShapeDtypeStruct((B, DIM), F32),
                   jax.ShapeDtypeStruct((B, REL * DIM), F32)],
    )(i0_pos, z1_pos, s2g, ihyper, m4i2, wgi1, rview)


def _hop_core(vh, h, t0, t1, item):
    jg = lax.broadcasted_iota(jnp.int32, (MD, NMEM), 0) // DIM
    mg = lax.broadcasted_iota(jnp.int32, (MD, NMEM), 1)
    G = (jg == mg).astype(F32)
    ji = lax.broadcasted_iota(jnp.int32, (MD, DIM), 0) % DIM
    ii = lax.broadcasted_iota(jnp.int32, (MD, DIM), 1)
    S = (ji == ii).astype(F32)
    hv = h * vh
    logits = jnp.dot(hv, G, preferred_element_type=F32)
    mx = jnp.max(logits, axis=1, keepdims=True)
    ex = jnp.exp(logits - mx)
    p = ex / jnp.sum(ex, axis=1, keepdims=True)
    pe = _dotT(p, G)
    w = (t0 + t1) * pe
    o = jnp.dot(w, S, preferred_element_type=F32)
    return o, item + o


_SPEC_H = pl.BlockSpec((1, B, MD), lambda i: (0, 0, 0))
_SPEC_H1 = pl.BlockSpec((1, B, MD), lambda i: (1, 0, 0))


def _t_spec(hop, ar):
    return pl.BlockSpec((1, 1, B, MD), lambda i: (hop, ar, 0, 0))


def _k6_hop0(vh, h_arr, t_arr, item, rview):
    def body(vh_r, h_r, t0_r, t1_r, it_r, rv_r, it1_o, o_o, v1_o):
        o, it1 = _hop_core(vh_r[...], h_r[0], t0_r[0, 0], t1_r[0, 0],
                           it_r[...])
        it1_o[...] = it1
        o_o[...] = o
        v1_o[...] = _dotT(it1, rv_r[...])

    return pl.pallas_call(
        body,
        grid=(1,),
        in_specs=[pl.BlockSpec((B, MD), lambda i: (0, 0)), _SPEC_H,
                  _t_spec(0, 0), _t_spec(0, 1),
                  pl.BlockSpec((B, DIM), lambda i: (0, 0)),
                  pl.BlockSpec((REL * DIM, DIM), lambda i: (0, 0))],
        out_specs=[pl.BlockSpec((B, DIM), lambda i: (0, 0)),
                   pl.BlockSpec((B, DIM), lambda i: (0, 0)),
                   pl.BlockSpec((B, REL * DIM), lambda i: (0, 0))],
        out_shape=[jax.ShapeDtypeStruct((B, DIM), F32),
                   jax.ShapeDtypeStruct((B, DIM), F32),
                   jax.ShapeDtypeStruct((B, REL * DIM), F32)],
    )(vh, h_arr, t_arr, t_arr, item, rview)


def _k6_hop1(vh, h_arr, t_arr, item1, o0):
    def body(vh_r, h_r, t0_r, t1_r, it_r, o0_r, sc_o):
        o1, it2 = _hop_core(vh_r[...], h_r[0], t0_r[0, 0], t1_r[0, 0],
                            it_r[...])
        y = o0_r[...] + o1
        s = jax.nn.sigmoid(jnp.sum(it2 * y, axis=1))
        sc_o[...] = s.reshape(8, 128)

    return pl.pallas_call(
        body,
        grid=(1,),
        in_specs=[pl.BlockSpec((B, MD), lambda i: (0, 0)), _SPEC_H1,
                  _t_spec(1, 0), _t_spec(1, 1),
                  pl.BlockSpec((B, DIM), lambda i: (0, 0)),
                  pl.BlockSpec((B, DIM), lambda i: (0, 0))],
        out_specs=pl.BlockSpec((8, 128), lambda i: (0, 0)),
        out_shape=jax.ShapeDtypeStruct((8, 128), F32),
    )(vh, h_arr, t_arr, t_arr, item1, o0)


def kernel(uEmbed0, iEmbed0, uhyper, ihyper, entity_emb, relation_emb,
           W_hgnn_u, W_hgnn_i, W_gcn_u, W_gcn_i, edge_vals,
           edge_index, pos_items, memories_h, memories_r, memories_t):
    rows = edge_index[0]
    cols = edge_index[1]
    u0p = uEmbed0.reshape(NB, PB, 128)
    i0p = iEmbed0.reshape(NB, PB, 128)
    u0v = u0p.reshape(U, DIM)
    i0v = i0p.reshape(I, DIM)

    h_flat = _sc_gather(entity_emb, memories_h.reshape(-1))
    t_flat = _sc_gather(entity_emb, memories_t.reshape(-1))
    i0_pos = _sc_gather(i0v, pos_items)
    h_arr = h_flat.reshape(NHOP, B, MD)
    t_arr = t_flat.reshape(NHOP, 2, B, MD)

    SU1, SI1 = _spmm_edges(u0v, i0v, rows, cols, both=True)
    SU1p = SU1.reshape(2, NB, PB, 128)
    SI1p = SI1.reshape(2, NB, PB, 128)
    m4u, m4i = _k2_l1(u0p, i0p, uhyper, ihyper, W_hgnn_u[0], W_hgnn_i[0])
    u1p, z1p = _k4_l1(u0p, i0p, uhyper, ihyper, m4u, m4i,
                      SU1p, SI1p, W_gcn_u[0], W_gcn_i[0])

    u1 = u1p.reshape(U, DIM)
    z1 = z1p.reshape(I, DIM)
    SI2 = _spmm_edges(u1, u1, rows, cols, both=False)
    m4i2 = _k2_l2(i0p, z1p, ihyper, W_hgnn_i[1])
    z1_pos = _sc_gather(z1, pos_items)
    s2_idx = jnp.concatenate([pos_items, pos_items + U])
    s2g = _sc_gather(SI2.reshape(2 * U, DIM), s2_idx)

    rview = relation_emb.reshape(REL * DIM, DIM)
    item_emb, v0 = _k_item(i0_pos, z1_pos, s2g, ihyper, m4i2,
                           W_gcn_i[1], rview)

    ridx = (memories_r.astype(I32)
            + jnp.arange(B, dtype=I32)[None, :, None] * REL)
    vh0 = _sc_gather(v0.reshape(B * REL, DIM), ridx[0].reshape(-1))
    item1, o0, v1 = _k6_hop0(vh0.reshape(B, MD), h_arr, t_arr,
                             item_emb, rview)
    vh1 = _sc_gather(v1.reshape(B * REL, DIM), ridx[1].reshape(-1))
    scores = _k6_hop1(vh1.reshape(B, MD), h_arr, t_arr, item1, o0)
    return scores.reshape(B)
